# baseline (speedup 1.0000x reference)
import functools

import jax
import jax.numpy as jnp
from jax import lax
from jax.experimental import pallas as pl
from jax.experimental.pallas import tpu as pltpu
from jax.experimental.pallas import tpu_sc as plsc

NC, NS, LANES = 2, 16, 16
NW = NC * NS
_N = 10000
_NP = 10240
_E = 320000
_D = 128
_K = 8
_H = 512
_O = 64

CHUNK = 80
NCHUNK = 128
EPT = NCHUNK * CHUNK
_EP = NW * EPT
RPT = _NP // NS

_mesh = plsc.VectorSubcoreMesh(
    core_axis_name="c", subcore_axis_name="s", num_cores=NC, num_subcores=NS)


def _make_hop(dim):
    @functools.partial(
        pl.kernel,
        out_type=jax.ShapeDtypeStruct((NC, _NP, dim), jnp.float32),
        mesh=_mesh,
        scratch_types=[
            pltpu.VMEM((EPT,), jnp.int32),
            pltpu.VMEM((NCHUNK, CHUNK), jnp.int32),
            pltpu.VMEM((2, CHUNK, dim), jnp.float32),
            pltpu.VMEM_SHARED((_NP, dim), jnp.float32),
            pltpu.SemaphoreType.DMA,
            pltpu.SemaphoreType.DMA,
            pltpu.SemaphoreType.DMA,
            pltpu.SemaphoreType.DMA,
        ],
    )
    def hop(y_hbm, srcr_hbm, dstr_hbm, out_hbm, sidx, didx, rows,
            acc_sh, gsem0, gsem1, ssem0, ssem1):
        c = lax.axis_index("c")
        s = lax.axis_index("s")
        widg = s * NC + c

        zero16 = jnp.zeros((LANES,), jnp.float32)
        lpr = max(dim // LANES, 1)

        def zbody(i, carry):
            r = i // lpr
            l = i % lpr
            rows[0, r, pl.ds(l * LANES, LANES)] = zero16
            return carry

        lax.fori_loop(0, CHUNK * lpr, zbody, 0)

        for k in range(RPT // CHUNK):
            pltpu.sync_copy(rows.at[0],
                            acc_sh.at[pl.ds(s * RPT + k * CHUNK, CHUNK)])
        plsc.subcore_barrier()

        pltpu.sync_copy(srcr_hbm.at[pl.ds(widg * EPT, EPT)], sidx)
        pltpu.sync_copy(dstr_hbm.at[widg], didx)

        ng = NCHUNK // 2

        def sref(j):
            return sidx.at[pl.ds(j * CHUNK, CHUNK)]

        pltpu.async_copy(y_hbm.at[sref(0)], rows.at[0], gsem0)
        pltpu.async_copy(y_hbm.at[sref(1)], rows.at[1], gsem1)

        def ebody(g, carry):
            j0 = g * 2
            j1 = j0 + 1
            pltpu.make_async_copy(y_hbm.at[sref(j0)], rows.at[0],
                                  gsem0).wait()
            pltpu.async_copy(rows.at[0], acc_sh.at[didx.at[j0]], ssem0,
                             add=True)
            pltpu.make_async_copy(y_hbm.at[sref(j1)], rows.at[1],
                                  gsem1).wait()
            pltpu.async_copy(rows.at[1], acc_sh.at[didx.at[j1]], ssem1,
                             add=True)

            @pl.when(g + 1 < ng)
            def _():
                pltpu.make_async_copy(rows.at[0], acc_sh.at[didx.at[j0]],
                                      ssem0).wait()
                pltpu.async_copy(y_hbm.at[sref(j0 + 2)], rows.at[0], gsem0)
                pltpu.make_async_copy(rows.at[1], acc_sh.at[didx.at[j1]],
                                      ssem1).wait()
                pltpu.async_copy(y_hbm.at[sref(j1 + 2)], rows.at[1], gsem1)

            return carry

        lax.fori_loop(0, ng, ebody, 0)
        pltpu.make_async_copy(rows.at[0], acc_sh.at[didx.at[NCHUNK - 2]],
                              ssem0).wait()
        pltpu.make_async_copy(rows.at[1], acc_sh.at[didx.at[NCHUNK - 1]],
                              ssem1).wait()
        plsc.subcore_barrier()

        for k in range(RPT // CHUNK):
            pltpu.sync_copy(acc_sh.at[pl.ds(s * RPT + k * CHUNK, CHUNK)],
                            out_hbm.at[c, pl.ds(s * RPT + k * CHUNK, CHUNK)])

    return hop


_hop128 = _make_hop(_D)

_BLK = 400


def _mlp_body(h_ref, w1_ref, b1_ref, sc_ref, bt_ref, w2_ref, b2_ref, o_ref):
    z = jnp.dot(h_ref[...], w1_ref[...], preferred_element_type=jnp.float32)
    z = z + b1_ref[...]
    a = jnp.where(z > 0, z, 0.2 * z)
    a = a * sc_ref[...] + bt_ref[...]
    o = jnp.dot(a, w2_ref[...], preferred_element_type=jnp.float32)
    o_ref[...] = o + b2_ref[...]


def _mlp_call(h, w1t, b1, scale, beta, w2t, b2):
    fan1 = _D * (_K + 1)
    return pl.pallas_call(
        _mlp_body,
        grid=(_N // _BLK,),
        in_specs=[
            pl.BlockSpec((_BLK, fan1), lambda i: (i, 0)),
            pl.BlockSpec((fan1, _H), lambda i: (0, 0)),
            pl.BlockSpec((1, _H), lambda i: (0, 0)),
            pl.BlockSpec((1, _H), lambda i: (0, 0)),
            pl.BlockSpec((1, _H), lambda i: (0, 0)),
            pl.BlockSpec((_H, _O), lambda i: (0, 0)),
            pl.BlockSpec((1, _O), lambda i: (0, 0)),
        ],
        out_specs=pl.BlockSpec((_BLK, _O), lambda i: (i, 0)),
        out_shape=jax.ShapeDtypeStruct((_N, _O), jnp.float32),
    )(h, w1t, b1.reshape(1, _H), scale.reshape(1, _H), beta.reshape(1, _H),
      w2t, b2.reshape(1, _O))


def kernel(feature, edge_index, W1, b1, gamma, beta, W2, b2):
    pad = jnp.full((_EP - _E,), _NP - 1, jnp.int32)
    srcr = jnp.concatenate([edge_index[0], pad])
    dstr = jnp.concatenate([edge_index[1], pad]).reshape(NW, NCHUNK, CHUNK)

    onest = jnp.ones((_NP, _D), jnp.float32)
    degp = _hop128(onest, srcr, dstr)
    deg = degp[0, :, 0] + degp[1, :, 0]
    dinv = jnp.where(deg > 0, lax.rsqrt(jnp.maximum(deg, 1e-12)), 0.0)
    dcol = dinv[:, None]

    featp = jnp.pad(feature, ((0, _NP - _N), (0, 0)))
    y = dcol * featp
    xs = [feature]
    for k in range(_K):
        p = _hop128(y, srcr, dstr)
        x = dcol * (p[0] + p[1])
        xs.append(x[:_N])
        if k < _K - 1:
            y = dcol * x

    h = jnp.concatenate(xs, axis=1)
    scale = gamma * (1.0 / jnp.sqrt(1.0 + 1e-5))
    return _mlp_call(h, W1.T, b1, scale, beta, W2.T, b2)


# speedup vs baseline: 1.1636x; 1.1636x over previous
import functools

import jax
import jax.numpy as jnp
from jax import lax
from jax.experimental import pallas as pl
from jax.experimental.pallas import tpu as pltpu
from jax.experimental.pallas import tpu_sc as plsc

NC, NS, LANES = 2, 16, 16
NW = NC * NS
_N = 10000
_NP = 10240
_E = 320000
_D = 128
_K = 8
_H = 512
_O = 64

CHUNK = 64
NCHUNK = 160
EPT = NCHUNK * CHUNK
_EP = NW * EPT
RPT = _NP // NS

_mesh = plsc.VectorSubcoreMesh(
    core_axis_name="c", subcore_axis_name="s", num_cores=NC, num_subcores=NS)


def _make_hop(dim):
    @functools.partial(
        pl.kernel,
        out_type=jax.ShapeDtypeStruct((NC, _NP, dim), jnp.float32),
        mesh=_mesh,
        scratch_types=[
            pltpu.VMEM((EPT,), jnp.int32),
            pltpu.VMEM((NCHUNK, CHUNK), jnp.int32),
            pltpu.VMEM((2, CHUNK, dim), jnp.float32),
            pltpu.VMEM_SHARED((_NP, dim), jnp.float32),
            pltpu.SemaphoreType.DMA,
            pltpu.SemaphoreType.DMA,
            pltpu.SemaphoreType.DMA,
            pltpu.SemaphoreType.DMA,
        ],
    )
    def hop(y_hbm, srcr_hbm, dstr_hbm, out_hbm, sidx, didx, rows,
            acc_sh, gsem0, gsem1, ssem0, ssem1):
        c = lax.axis_index("c")
        s = lax.axis_index("s")
        widg = s * NC + c

        zero16 = jnp.zeros((LANES,), jnp.float32)
        lpr = max(dim // LANES, 1)

        def zbody(i, carry):
            r = i // lpr
            l = i % lpr
            rows[0, r, pl.ds(l * LANES, LANES)] = zero16
            return carry

        lax.fori_loop(0, CHUNK * lpr, zbody, 0)

        for k in range(RPT // CHUNK):
            pltpu.sync_copy(rows.at[0],
                            acc_sh.at[pl.ds(s * RPT + k * CHUNK, CHUNK)])
        plsc.subcore_barrier()

        pltpu.sync_copy(srcr_hbm.at[pl.ds(widg * EPT, EPT)], sidx)
        pltpu.sync_copy(dstr_hbm.at[widg], didx)

        ng = NCHUNK // 2

        def sref(j):
            return sidx.at[pl.ds(j * CHUNK, CHUNK)]

        pltpu.async_copy(y_hbm.at[sref(0)], rows.at[0], gsem0)
        pltpu.async_copy(y_hbm.at[sref(1)], rows.at[1], gsem1)

        def ebody(g, carry):
            j0 = g * 2
            j1 = j0 + 1
            pltpu.make_async_copy(y_hbm.at[sref(j0)], rows.at[0],
                                  gsem0).wait()
            pltpu.async_copy(rows.at[0], acc_sh.at[didx.at[j0]], ssem0,
                             add=True)
            pltpu.make_async_copy(y_hbm.at[sref(j1)], rows.at[1],
                                  gsem1).wait()
            pltpu.async_copy(rows.at[1], acc_sh.at[didx.at[j1]], ssem1,
                             add=True)

            @pl.when(g + 1 < ng)
            def _():
                pltpu.make_async_copy(rows.at[0], acc_sh.at[didx.at[j0]],
                                      ssem0).wait()
                pltpu.async_copy(y_hbm.at[sref(j0 + 2)], rows.at[0], gsem0)
                pltpu.make_async_copy(rows.at[1], acc_sh.at[didx.at[j1]],
                                      ssem1).wait()
                pltpu.async_copy(y_hbm.at[sref(j1 + 2)], rows.at[1], gsem1)

            return carry

        lax.fori_loop(0, ng, ebody, 0)
        pltpu.make_async_copy(rows.at[0], acc_sh.at[didx.at[NCHUNK - 2]],
                              ssem0).wait()
        pltpu.make_async_copy(rows.at[1], acc_sh.at[didx.at[NCHUNK - 1]],
                              ssem1).wait()
        plsc.subcore_barrier()

        for k in range(RPT // CHUNK):
            pltpu.sync_copy(acc_sh.at[pl.ds(s * RPT + k * CHUNK, CHUNK)],
                            out_hbm.at[c, pl.ds(s * RPT + k * CHUNK, CHUNK)])

    return hop


_hop128 = _make_hop(_D)

_BLK = 400


def _mlp_body(h_ref, w1_ref, b1_ref, sc_ref, bt_ref, w2_ref, b2_ref, o_ref):
    z = jnp.dot(h_ref[...], w1_ref[...], preferred_element_type=jnp.float32)
    z = z + b1_ref[...]
    a = jnp.where(z > 0, z, 0.2 * z)
    a = a * sc_ref[...] + bt_ref[...]
    o = jnp.dot(a, w2_ref[...], preferred_element_type=jnp.float32)
    o_ref[...] = o + b2_ref[...]


def _mlp_call(h, w1t, b1, scale, beta, w2t, b2):
    fan1 = _D * (_K + 1)
    return pl.pallas_call(
        _mlp_body,
        grid=(_N // _BLK,),
        in_specs=[
            pl.BlockSpec((_BLK, fan1), lambda i: (i, 0)),
            pl.BlockSpec((fan1, _H), lambda i: (0, 0)),
            pl.BlockSpec((1, _H), lambda i: (0, 0)),
            pl.BlockSpec((1, _H), lambda i: (0, 0)),
            pl.BlockSpec((1, _H), lambda i: (0, 0)),
            pl.BlockSpec((_H, _O), lambda i: (0, 0)),
            pl.BlockSpec((1, _O), lambda i: (0, 0)),
        ],
        out_specs=pl.BlockSpec((_BLK, _O), lambda i: (i, 0)),
        out_shape=jax.ShapeDtypeStruct((_N, _O), jnp.float32),
    )(h, w1t, b1.reshape(1, _H), scale.reshape(1, _H), beta.reshape(1, _H),
      w2t, b2.reshape(1, _O))


def kernel(feature, edge_index, W1, b1, gamma, beta, W2, b2):
    pad = jnp.full((_EP - _E,), _NP - 1, jnp.int32)
    srcr = jnp.concatenate([edge_index[0], pad])
    dstr = jnp.concatenate([edge_index[1], pad]).reshape(NW, NCHUNK, CHUNK)

    onest = jnp.ones((_NP, _D), jnp.float32)
    degp = _hop128(onest, srcr, dstr)
    deg = degp[0, :, 0] + degp[1, :, 0]
    dinv = jnp.where(deg > 0, lax.rsqrt(jnp.maximum(deg, 1e-12)), 0.0)
    dcol = dinv[:, None]

    featp = jnp.pad(feature, ((0, _NP - _N), (0, 0)))
    y = dcol * featp
    xs = [feature]
    for k in range(_K):
        p = _hop128(y, srcr, dstr)
        x = dcol * (p[0] + p[1])
        xs.append(x[:_N])
        if k < _K - 1:
            y = dcol * x

    h = jnp.concatenate(xs, axis=1)
    scale = gamma * (1.0 / jnp.sqrt(1.0 + 1e-5))
    return _mlp_call(h, W1.T, b1, scale, beta, W2.T, b2)
